# baseline (speedup 1.0000x reference)
import functools

import jax
import jax.numpy as jnp
from jax import lax
from jax.experimental import pallas as pl
from jax.experimental.pallas import tpu as pltpu
from jax.experimental.pallas import tpu_sc as plsc

B = 16384
D = 32
L = 16

_info = plsc.get_sparse_core_info()
NC = _info.num_cores
NS = _info.num_subcores
NW = NC * NS
B_PER_W = B // NW
GCHUNK = 128
NCHUNKS = B_PER_W // GCHUNK


def _body(uid_hbm, mid_hbm, ut_hbm, mt_hbm, out_hbm,
          idxu_v, idxm_v, urows_v, mrows_v, hb_v, outv_v, sem_u, sem_m):
    wid = lax.axis_index("s") * NC + lax.axis_index("c")
    base = wid * B_PER_W

    pltpu.sync_copy(uid_hbm.at[pl.ds(base, B_PER_W)], idxu_v)
    pltpu.sync_copy(mid_hbm.at[pl.ds(base, B_PER_W)], idxm_v)

    def fire(c, _):
        s = c * 16
        idx_u = idxu_v[pl.ds(s, 16)]
        pltpu.async_copy(ut_hbm.at[idx_u], urows_v.at[pl.ds(s, 16), :], sem_u)
        idx_m = idxm_v[pl.ds(s, 16)]
        pltpu.async_copy(mt_hbm.at[idx_m], mrows_v.at[pl.ds(s, 16), :], sem_m)
        return 0

    lax.fori_loop(0, B_PER_W // 16, fire, 0)
    pltpu.make_async_copy(ut_hbm.at[pl.ds(0, B_PER_W), :], urows_v, sem_u).wait()
    pltpu.make_async_copy(mt_hbm.at[pl.ds(0, B_PER_W), :], mrows_v, sem_m).wait()

    lanes16 = lax.iota(jnp.int32, 16) * 16

    def block(b, _):
        rbase = b * 16

        def prow(r, _):
            row = rbase + r
            p = (urows_v[row, pl.ds(0, 16)] * mrows_v[row, pl.ds(0, 16)]
                 + urows_v[row, pl.ds(16, 16)] * mrows_v[row, pl.ds(16, 16)])
            hb_v[pl.ds(r * 16, 16)] = p
            return 0

        lax.fori_loop(0, 16, prow, 0)

        def tsum(j, acc):
            return acc + plsc.load_gather(hb_v, [lanes16 + j])

        outv_v[pl.ds(rbase, 16)] = lax.fori_loop(
            0, 16, tsum, jnp.zeros((16,), jnp.float32))
        return 0

    lax.fori_loop(0, B_PER_W // 16, block, 0)

    pltpu.sync_copy(outv_v, out_hbm.at[pl.ds(base, B_PER_W)])


@jax.jit
def _run(user_ids, movie_ids, user_table, movie_table):
    mesh = plsc.VectorSubcoreMesh(core_axis_name="c", subcore_axis_name="s")
    k = pl.kernel(
        _body,
        mesh=mesh,
        out_type=jax.ShapeDtypeStruct((B,), jnp.float32),
        scratch_types=[
            pltpu.VMEM((B_PER_W,), jnp.int32),
            pltpu.VMEM((B_PER_W,), jnp.int32),
            pltpu.VMEM((B_PER_W, D), jnp.float32),
            pltpu.VMEM((B_PER_W, D), jnp.float32),
            pltpu.VMEM((16 * 16,), jnp.float32),
            pltpu.VMEM((B_PER_W,), jnp.float32),
            pltpu.SemaphoreType.DMA,
            pltpu.SemaphoreType.DMA,
        ],
        compiler_params=pltpu.CompilerParams(
            needs_layout_passes=False, use_tc_tiling_on_sc=False),
    )
    return k(user_ids, movie_ids, user_table, movie_table)


def kernel(user_ids, movie_ids, user_table, movie_table):
    return _run(user_ids, movie_ids, user_table, movie_table)


# speedup vs baseline: 1.0001x; 1.0001x over previous
import jax
import jax.numpy as jnp
from jax import lax
from jax.experimental import pallas as pl
from jax.experimental.pallas import tpu as pltpu
from jax.experimental.pallas import tpu_sc as plsc

B = 16384
D = 32

_info = plsc.get_sparse_core_info()
NC = _info.num_cores
NS = _info.num_subcores
NW = NC * NS
B_PER_W = B // NW


def _body(uid_hbm, mid_hbm, ut_hbm, mt_hbm, out_hbm,
          idxu_v, idxm_v, urows_v, mrows_v, hb_v, outv_v, sem_u, sem_m):
    wid = lax.axis_index("s") * NC + lax.axis_index("c")
    base = wid * B_PER_W

    pltpu.sync_copy(uid_hbm.at[pl.ds(base, B_PER_W)], idxu_v)
    pltpu.sync_copy(mid_hbm.at[pl.ds(base, B_PER_W)], idxm_v)

    thirty_two = jnp.int32(32)

    def fire(c, _):
        s = c * 16
        vu = idxu_v[pl.ds(s, 16)] * thirty_two
        vm = idxm_v[pl.ds(s, 16)] * thirty_two
        for j in range(16):
            ou = pl.multiple_of(vu[j], 32)
            om = pl.multiple_of(vm[j], 32)
            pltpu.async_copy(ut_hbm.at[pl.ds(ou, D)],
                             urows_v.at[pl.ds((s + j) * D, D)], sem_u)
            pltpu.async_copy(mt_hbm.at[pl.ds(om, D)],
                             mrows_v.at[pl.ds((s + j) * D, D)], sem_m)
        return 0

    lax.fori_loop(0, B_PER_W // 16, fire, 0)
    pltpu.make_async_copy(ut_hbm.at[pl.ds(0, B_PER_W * D)], urows_v, sem_u).wait()
    pltpu.make_async_copy(mt_hbm.at[pl.ds(0, B_PER_W * D)], mrows_v, sem_m).wait()

    lanes16 = lax.iota(jnp.int32, 16) * 16

    def block(b, _):
        rbase = b * 16

        def prow(r, _):
            row = (rbase + r) * D
            p = (urows_v[pl.ds(row, 16)] * mrows_v[pl.ds(row, 16)]
                 + urows_v[pl.ds(row + 16, 16)] * mrows_v[pl.ds(row + 16, 16)])
            hb_v[pl.ds(r * 16, 16)] = p
            return 0

        lax.fori_loop(0, 16, prow, 0)

        def tsum(j, acc):
            return acc + plsc.load_gather(hb_v, [lanes16 + j])

        outv_v[pl.ds(rbase, 16)] = lax.fori_loop(
            0, 16, tsum, jnp.zeros((16,), jnp.float32))
        return 0

    lax.fori_loop(0, B_PER_W // 16, block, 0)
    pltpu.sync_copy(outv_v, out_hbm.at[pl.ds(base, B_PER_W)])


@jax.jit
def _run(user_ids, movie_ids, ut, mt):
    mesh = plsc.VectorSubcoreMesh(core_axis_name="c", subcore_axis_name="s")
    k = pl.kernel(
        _body,
        mesh=mesh,
        out_type=jax.ShapeDtypeStruct((B,), jnp.float32),
        scratch_types=[
            pltpu.VMEM((B_PER_W,), jnp.int32),
            pltpu.VMEM((B_PER_W,), jnp.int32),
            pltpu.VMEM((B_PER_W * D,), jnp.float32),
            pltpu.VMEM((B_PER_W * D,), jnp.float32),
            pltpu.VMEM((16 * 16,), jnp.float32),
            pltpu.VMEM((B_PER_W,), jnp.float32),
            pltpu.SemaphoreType.DMA,
            pltpu.SemaphoreType.DMA,
        ],
        compiler_params=pltpu.CompilerParams(
            needs_layout_passes=False, use_tc_tiling_on_sc=False),
    )
    return k(user_ids, movie_ids, ut, mt)


def kernel(user_ids, movie_ids, user_table, movie_table):
    return _run(user_ids, movie_ids,
                user_table.reshape(-1), movie_table.reshape(-1))


# speedup vs baseline: 1.0051x; 1.0050x over previous
import jax
import jax.numpy as jnp
from jax import lax
from jax.experimental import pallas as pl
from jax.experimental.pallas import tpu as pltpu
from jax.experimental.pallas import tpu_sc as plsc

B = 16384
D = 32
DP = 128

_info = plsc.get_sparse_core_info()
NC = _info.num_cores
NS = _info.num_subcores
NW = NC * NS
B_PER_W = B // NW


def _body(uid_hbm, mid_hbm, ut_hbm, mt_hbm, out_hbm,
          idxu_v, idxm_v, urows_v, mrows_v, hb_v, outv_v, sem_u, sem_m):
    wid = lax.axis_index("s") * NC + lax.axis_index("c")
    base = wid * B_PER_W

    pltpu.sync_copy(uid_hbm.at[pl.ds(base, B_PER_W)], idxu_v)
    pltpu.sync_copy(mid_hbm.at[pl.ds(base, B_PER_W)], idxm_v)

    lanes16 = lax.iota(jnp.int32, 16) * 16
    HALF = B_PER_W // 2

    def chunk(h, _):
        hbase = h * HALF

        def fire(c, _):
            s = hbase + c * 16
            vu = idxu_v[pl.ds(s, 16)]
            pltpu.async_copy(ut_hbm.at[vu],
                             urows_v.at[pl.ds(c * 16, 16), :], sem_u)
            vm = idxm_v[pl.ds(s, 16)]
            pltpu.async_copy(mt_hbm.at[vm],
                             mrows_v.at[pl.ds(c * 16, 16), :], sem_m)
            return 0

        lax.fori_loop(0, HALF // 16, fire, 0)
        pltpu.make_async_copy(ut_hbm.at[pl.ds(0, HALF), :], urows_v, sem_u).wait()
        pltpu.make_async_copy(mt_hbm.at[pl.ds(0, HALF), :], mrows_v, sem_m).wait()

        def block(b, _):
            rbase = b * 16

            def prow(r, _):
                row = rbase + r
                p = (urows_v[row, pl.ds(0, 16)] * mrows_v[row, pl.ds(0, 16)]
                     + urows_v[row, pl.ds(16, 16)] * mrows_v[row, pl.ds(16, 16)])
                hb_v[pl.ds(r * 16, 16)] = p
                return 0

            lax.fori_loop(0, 16, prow, 0)

            def tsum(j, acc):
                return acc + plsc.load_gather(hb_v, [lanes16 + j])

            outv_v[pl.ds(hbase + rbase, 16)] = lax.fori_loop(
                0, 16, tsum, jnp.zeros((16,), jnp.float32))
            return 0

        lax.fori_loop(0, HALF // 16, block, 0)
        return 0

    lax.fori_loop(0, 2, chunk, 0)
    pltpu.sync_copy(outv_v, out_hbm.at[pl.ds(base, B_PER_W)])


@jax.jit
def _run(user_ids, movie_ids, ut, mt):
    mesh = plsc.VectorSubcoreMesh(core_axis_name="c", subcore_axis_name="s")
    k = pl.kernel(
        _body,
        mesh=mesh,
        out_type=jax.ShapeDtypeStruct((B,), jnp.float32),
        scratch_types=[
            pltpu.VMEM((B_PER_W,), jnp.int32),
            pltpu.VMEM((B_PER_W,), jnp.int32),
            pltpu.VMEM((B_PER_W // 2, DP), jnp.float32),
            pltpu.VMEM((B_PER_W // 2, DP), jnp.float32),
            pltpu.VMEM((16 * 16,), jnp.float32),
            pltpu.VMEM((B_PER_W,), jnp.float32),
            pltpu.SemaphoreType.DMA,
            pltpu.SemaphoreType.DMA,
        ],
        compiler_params=pltpu.CompilerParams(
            needs_layout_passes=False, use_tc_tiling_on_sc=False),
    )
    return k(user_ids, movie_ids, ut, mt)


def kernel(user_ids, movie_ids, user_table, movie_table):
    ut = jnp.pad(user_table, ((0, 0), (0, DP - D)))
    mt = jnp.pad(movie_table, ((0, 0), (0, DP - D)))
    return _run(user_ids, movie_ids, ut, mt)
